# speedup vs baseline: 32.5358x; 32.5358x over previous
import functools

import jax
import jax.numpy as jnp
from jax import lax
from jax.experimental import pallas as pl
from jax.experimental.pallas import tpu as pltpu
from jax.experimental.pallas import tpu_sc as plsc

N_NODE = 320000
N_BATCH = 10000
OUT_DIM = 128
N_ATOMREF = 100
HIST_W = 128

NC = 2
NS = 16
NW = NC * NS
ATOMS_PER_TILE = N_NODE // NW
VECS = ATOMS_PER_TILE // 16
IDX_ROWS = 79
HIST_WORDS = N_BATCH * HIST_W
WORDS_PER_TILE = HIST_WORDS // NS
ZCHUNK = 10000


def _hist_body(z_hbm, b_hbm, hist_hbm, shared, zv, bv, fidx, vfull, vlast, zbuf):
    c = lax.axis_index("c")
    s = lax.axis_index("s")
    wid = c * NS + s
    base = wid * ATOMS_PER_TILE

    pltpu.sync_copy(z_hbm.at[pl.ds(base, ATOMS_PER_TILE)], zv)
    pltpu.sync_copy(b_hbm.at[pl.ds(base, ATOMS_PER_TILE)], bv)

    zeros_f = jnp.zeros((16,), jnp.float32)
    ones_f = jnp.ones((16,), jnp.float32)
    zeros_i = jnp.zeros((16,), jnp.int32)

    def _zb(i, carry):
        zbuf[pl.ds(i * 16, 16)] = zeros_f
        return carry
    lax.fori_loop(0, ZCHUNK // 16, _zb, 0)
    for k in range(WORDS_PER_TILE // ZCHUNK):
        pltpu.sync_copy(
            zbuf, shared.at[pl.ds(s * WORDS_PER_TILE + k * ZCHUNK, ZCHUNK)])

    for k in range(8):
        vfull[pl.ds(k * 16, 16)] = ones_f
        vlast[pl.ds(k * 16, 16)] = zeros_f
    vlast[pl.ds(0, 16)] = ones_f

    def _fb(i, carry):
        r = i // 8
        col = (i % 8) * 16

        @pl.when(i < VECS)
        def _():
            off = i * 16
            zi = zv[pl.ds(off, 16)]
            bi = bv[pl.ds(off, 16)]
            fidx[r, pl.ds(col, 16)] = bi * HIST_W + zi

        @pl.when(i >= VECS)
        def _():
            fidx[r, pl.ds(col, 16)] = zeros_i

        return carry
    lax.fori_loop(0, IDX_ROWS * 8, _fb, 0)

    plsc.subcore_barrier()

    def _sc(j, carry):
        pltpu.sync_copy(vfull, shared.at[fidx.at[j]], add=True)
        return carry
    lax.fori_loop(0, IDX_ROWS - 1, _sc, 0)
    pltpu.sync_copy(vlast, shared.at[fidx.at[IDX_ROWS - 1]], add=True)

    plsc.subcore_barrier()

    pltpu.sync_copy(
        shared.at[pl.ds(s * WORDS_PER_TILE, WORDS_PER_TILE)],
        hist_hbm.at[c, pl.ds(s * WORDS_PER_TILE, WORDS_PER_TILE)])


_hist = functools.partial(
    pl.kernel,
    out_type=jax.ShapeDtypeStruct((NC, HIST_WORDS), jnp.float32),
    mesh=plsc.VectorSubcoreMesh(core_axis_name="c", subcore_axis_name="s"),
    scratch_types=[
        pltpu.VMEM_SHARED((HIST_WORDS,), jnp.float32),
        pltpu.VMEM((ATOMS_PER_TILE,), jnp.int32),
        pltpu.VMEM((ATOMS_PER_TILE,), jnp.int32),
        pltpu.VMEM((IDX_ROWS, 128), jnp.int32),
        pltpu.VMEM((128,), jnp.float32),
        pltpu.VMEM((128,), jnp.float32),
        pltpu.VMEM((ZCHUNK,), jnp.float32),
    ],
)(_hist_body)


ROWS_TC = 400
GRID_TC = N_BATCH // ROWS_TC


def _combine_body(h_ref, out_ref, table_ref, mean_ref, o_ref):
    h = h_ref[0] + h_ref[1]
    table = table_ref[...] + mean_ref[...]
    o_ref[...] = out_ref[...] + jnp.dot(
        h, table, preferred_element_type=jnp.float32)


def _combine(hist, out, table, mean2d):
    return pl.pallas_call(
        _combine_body,
        grid=(GRID_TC,),
        in_specs=[
            pl.BlockSpec((NC, ROWS_TC, HIST_W), lambda i: (0, i, 0)),
            pl.BlockSpec((ROWS_TC, OUT_DIM), lambda i: (i, 0)),
            pl.BlockSpec((HIST_W, OUT_DIM), lambda i: (0, 0)),
            pl.BlockSpec((1, OUT_DIM), lambda i: (0, 0)),
        ],
        out_specs=pl.BlockSpec((ROWS_TC, OUT_DIM), lambda i: (i, 0)),
        out_shape=jax.ShapeDtypeStruct((N_BATCH, OUT_DIM), jnp.float32),
    )(hist, out, table, mean2d)


def kernel(out, z, batch_idx, atomref, mean):
    hist = _hist(z, batch_idx).reshape(NC, N_BATCH, HIST_W)
    table = jnp.pad(atomref, ((0, HIST_W - N_ATOMREF), (0, 0)))
    return _combine(hist, out, table, mean.reshape(1, OUT_DIM))


# speedup vs baseline: 38.4839x; 1.1828x over previous
import functools

import jax
import jax.numpy as jnp
from jax import lax
from jax.experimental import pallas as pl
from jax.experimental.pallas import tpu as pltpu
from jax.experimental.pallas import tpu_sc as plsc

N_NODE = 320000
N_BATCH = 10000
OUT_DIM = 128
N_ATOMREF = 100
HIST_W = 128

NC = 2
NS = 16
NW = NC * NS
ATOMS_PER_TILE = N_NODE // NW
VECS = ATOMS_PER_TILE // 16
IDX_ROWS = 79
HIST_WORDS = N_BATCH * HIST_W
WORDS_PER_TILE = HIST_WORDS // NS
ZCHUNK = 10000


def _hist_body(z_hbm, b_hbm, hist_hbm, shared, zv, bv, fidx, vfull, vlast, zbuf):
    c = lax.axis_index("c")
    s = lax.axis_index("s")
    wid = c * NS + s
    base = wid * ATOMS_PER_TILE

    pltpu.sync_copy(z_hbm.at[pl.ds(base, ATOMS_PER_TILE)], zv)
    pltpu.sync_copy(b_hbm.at[pl.ds(base, ATOMS_PER_TILE)], bv)

    zeros_f = jnp.zeros((16,), jnp.float32)
    ones_f = jnp.ones((16,), jnp.float32)
    zeros_i = jnp.zeros((16,), jnp.int32)

    def _zb(i, carry):
        zbuf[pl.ds(i * 16, 16)] = zeros_f
        return carry
    lax.fori_loop(0, ZCHUNK // 16, _zb, 0)
    for k in range(WORDS_PER_TILE // ZCHUNK):
        pltpu.sync_copy(
            zbuf, shared.at[pl.ds(s * WORDS_PER_TILE + k * ZCHUNK, ZCHUNK)])

    for k in range(8):
        vfull[pl.ds(k * 16, 16)] = ones_f
        vlast[pl.ds(k * 16, 16)] = zeros_f
    vlast[pl.ds(0, 16)] = ones_f

    def _fb(i, carry):
        r = i // 8
        col = (i % 8) * 16

        @pl.when(i < VECS)
        def _():
            off = i * 16
            zi = zv[pl.ds(off, 16)]
            bi = bv[pl.ds(off, 16)]
            fidx[r, pl.ds(col, 16)] = bi * HIST_W + zi

        @pl.when(i >= VECS)
        def _():
            fidx[r, pl.ds(col, 16)] = zeros_i

        return carry
    lax.fori_loop(0, IDX_ROWS * 8, _fb, 0)

    plsc.subcore_barrier()

    def _sc(j, carry):
        pltpu.sync_copy(vfull, shared.at[fidx.at[j]], add=True)
        return carry
    lax.fori_loop(0, IDX_ROWS - 1, _sc, 0)
    pltpu.sync_copy(vlast, shared.at[fidx.at[IDX_ROWS - 1]], add=True)

    plsc.subcore_barrier()

    pltpu.sync_copy(
        shared.at[pl.ds(s * WORDS_PER_TILE, WORDS_PER_TILE)],
        hist_hbm.at[pl.ds(c * HIST_WORDS + s * WORDS_PER_TILE,
                          WORDS_PER_TILE)])


_hist = functools.partial(
    pl.kernel,
    out_type=jax.ShapeDtypeStruct((NC * HIST_WORDS,), jnp.float32),
    mesh=plsc.VectorSubcoreMesh(core_axis_name="c", subcore_axis_name="s"),
    scratch_types=[
        pltpu.VMEM_SHARED((HIST_WORDS,), jnp.float32),
        pltpu.VMEM((ATOMS_PER_TILE,), jnp.int32),
        pltpu.VMEM((ATOMS_PER_TILE,), jnp.int32),
        pltpu.VMEM((IDX_ROWS, 128), jnp.int32),
        pltpu.VMEM((128,), jnp.float32),
        pltpu.VMEM((128,), jnp.float32),
        pltpu.VMEM((ZCHUNK,), jnp.float32),
    ],
)(_hist_body)


ROWS_TC = 400
GRID_TC = N_BATCH // ROWS_TC


def _combine_body(h_ref, out_ref, table_ref, mean_ref, o_ref):
    h = h_ref[0] + h_ref[1]
    table = table_ref[...] + mean_ref[...]
    o_ref[...] = out_ref[...] + jnp.dot(
        h, table, preferred_element_type=jnp.float32)


def _combine(hist, out, table, mean2d):
    return pl.pallas_call(
        _combine_body,
        grid=(GRID_TC,),
        in_specs=[
            pl.BlockSpec((NC, ROWS_TC, HIST_W), lambda i: (0, i, 0)),
            pl.BlockSpec((ROWS_TC, OUT_DIM), lambda i: (i, 0)),
            pl.BlockSpec((HIST_W, OUT_DIM), lambda i: (0, 0)),
            pl.BlockSpec((1, OUT_DIM), lambda i: (0, 0)),
        ],
        out_specs=pl.BlockSpec((ROWS_TC, OUT_DIM), lambda i: (i, 0)),
        out_shape=jax.ShapeDtypeStruct((N_BATCH, OUT_DIM), jnp.float32),
    )(hist, out, table, mean2d)


def kernel(out, z, batch_idx, atomref, mean):
    hist = _hist(z, batch_idx).reshape(NC, N_BATCH, HIST_W)
    table = jnp.pad(atomref, ((0, HIST_W - N_ATOMREF), (0, 0)))
    return _combine(hist, out, table, mean.reshape(1, OUT_DIM))


# speedup vs baseline: 44.4259x; 1.1544x over previous
import functools

import jax
import jax.numpy as jnp
from jax import lax
from jax.experimental import pallas as pl
from jax.experimental.pallas import tpu as pltpu
from jax.experimental.pallas import tpu_sc as plsc

N_NODE = 320000
N_BATCH = 10000
OUT_DIM = 128
N_ATOMREF = 100
HIST_W = 128

NS = 16
NW = 2 * NS
ATOMS_PER_TILE = N_NODE // NW
VECS = ATOMS_PER_TILE // 16
IDX_ROWS = (VECS + 7) // 8
HIST_WORDS = N_BATCH * HIST_W
WORDS_PER_TILE = HIST_WORDS // NS
ZCHUNK = 10000
QLAG = 16
TRASH = HIST_WORDS


def _hist_body(z_hbm, b_hbm, h0_hbm, h1_hbm, shared, zv, bv, fidx, vfull,
               zbuf, sem_in, sem_z, sem_s):
    c = lax.axis_index("c")
    s = lax.axis_index("s")
    wid = c * NS + s
    base = wid * ATOMS_PER_TILE

    pltpu.async_copy(z_hbm.at[pl.ds(base, ATOMS_PER_TILE)], zv, sem_in)
    pltpu.async_copy(b_hbm.at[pl.ds(base, ATOMS_PER_TILE)], bv, sem_in)

    zeros_f = jnp.zeros((16,), jnp.float32)
    ones_f = jnp.ones((16,), jnp.float32)
    trash_i = jnp.full((16,), TRASH, jnp.int32)

    def _zb(i, carry):
        zbuf[pl.ds(i * 16, 16)] = zeros_f
        return carry
    lax.fori_loop(0, ZCHUNK // 16, _zb, 0)
    nz = WORDS_PER_TILE // ZCHUNK
    for k in range(nz):
        pltpu.async_copy(
            zbuf, shared.at[pl.ds(s * WORDS_PER_TILE + k * ZCHUNK, ZCHUNK)],
            sem_z)

    for k in range(8):
        vfull[pl.ds(k * 16, 16)] = ones_f

    pltpu.make_async_copy(z_hbm.at[pl.ds(base, ATOMS_PER_TILE)], zv,
                          sem_in).wait()
    pltpu.make_async_copy(b_hbm.at[pl.ds(base, ATOMS_PER_TILE)], bv,
                          sem_in).wait()

    def _fb(i, carry):
        r = i // 8
        col = (i % 8) * 16

        @pl.when(i < VECS)
        def _():
            off = i * 16
            zi = zv[pl.ds(off, 16)]
            bi = bv[pl.ds(off, 16)]
            fidx[r, pl.ds(col, 16)] = bi * HIST_W + zi

        @pl.when(i >= VECS)
        def _():
            fidx[r, pl.ds(col, 16)] = trash_i

        return carry
    lax.fori_loop(0, IDX_ROWS * 8, _fb, 0)

    for k in range(nz):
        pltpu.make_async_copy(
            zbuf, shared.at[pl.ds(s * WORDS_PER_TILE + k * ZCHUNK, ZCHUNK)],
            sem_z).wait()
    plsc.subcore_barrier()

    def _step(j, carry):
        @pl.when(j < IDX_ROWS)
        def _():
            pltpu.async_copy(vfull, shared.at[fidx.at[j]], sem_s, add=True)

        @pl.when(j >= QLAG)
        def _():
            pltpu.make_async_copy(vfull, shared.at[fidx.at[j - QLAG]],
                                  sem_s).wait()

        return carry
    lax.fori_loop(0, IDX_ROWS + QLAG, _step, 0)

    plsc.subcore_barrier()

    src = shared.at[pl.ds(s * WORDS_PER_TILE, WORDS_PER_TILE)]

    @pl.when(c == 0)
    def _():
        pltpu.sync_copy(src, h0_hbm.at[pl.ds(s * WORDS_PER_TILE,
                                             WORDS_PER_TILE)])

    @pl.when(c == 1)
    def _():
        pltpu.sync_copy(src, h1_hbm.at[pl.ds(s * WORDS_PER_TILE,
                                             WORDS_PER_TILE)])


_hist = functools.partial(
    pl.kernel,
    out_type=(jax.ShapeDtypeStruct((HIST_WORDS,), jnp.float32),
              jax.ShapeDtypeStruct((HIST_WORDS,), jnp.float32)),
    mesh=plsc.VectorSubcoreMesh(core_axis_name="c", subcore_axis_name="s"),
    scratch_types=[
        pltpu.VMEM_SHARED((HIST_WORDS + 128,), jnp.float32),
        pltpu.VMEM((ATOMS_PER_TILE,), jnp.int32),
        pltpu.VMEM((ATOMS_PER_TILE,), jnp.int32),
        pltpu.VMEM((IDX_ROWS, 128), jnp.int32),
        pltpu.VMEM((128,), jnp.float32),
        pltpu.VMEM((ZCHUNK,), jnp.float32),
        pltpu.SemaphoreType.DMA,
        pltpu.SemaphoreType.DMA,
        pltpu.SemaphoreType.DMA,
    ],
)(_hist_body)


ROWS_TC = 400
GRID_TC = N_BATCH // ROWS_TC


def _combine_body(h0_ref, h1_ref, out_ref, table_ref, mean_ref, o_ref):
    h = h0_ref[...] + h1_ref[...]
    table = table_ref[...] + mean_ref[...]
    o_ref[...] = out_ref[...] + jnp.dot(
        h, table, preferred_element_type=jnp.float32)


def _combine(h0, h1, out, table, mean2d):
    hspec = pl.BlockSpec((ROWS_TC, HIST_W), lambda i: (i, 0))
    return pl.pallas_call(
        _combine_body,
        grid=(GRID_TC,),
        in_specs=[
            hspec,
            hspec,
            pl.BlockSpec((ROWS_TC, OUT_DIM), lambda i: (i, 0)),
            pl.BlockSpec((HIST_W, OUT_DIM), lambda i: (0, 0)),
            pl.BlockSpec((1, OUT_DIM), lambda i: (0, 0)),
        ],
        out_specs=pl.BlockSpec((ROWS_TC, OUT_DIM), lambda i: (i, 0)),
        out_shape=jax.ShapeDtypeStruct((N_BATCH, OUT_DIM), jnp.float32),
    )(h0, h1, out, table, mean2d)


def kernel(out, z, batch_idx, atomref, mean):
    h0, h1 = _hist(z, batch_idx)
    h0 = h0.reshape(N_BATCH, HIST_W)
    h1 = h1.reshape(N_BATCH, HIST_W)
    table = jnp.pad(atomref, ((0, HIST_W - N_ATOMREF), (0, 0)))
    return _combine(h0, h1, out, table, mean.reshape(1, OUT_DIM))


# speedup vs baseline: 46.9532x; 1.0569x over previous
import functools

import jax
import jax.numpy as jnp
from jax import lax
from jax.experimental import pallas as pl
from jax.experimental.pallas import tpu as pltpu
from jax.experimental.pallas import tpu_sc as plsc

N_NODE = 320000
N_BATCH = 10000
OUT_DIM = 128
N_ATOMREF = 100
HIST_W = 128

NS = 16
NW = 2 * NS
ATOMS_PER_TILE = N_NODE // NW
VECS = ATOMS_PER_TILE // 16
IDX_ROWS = (VECS + 7) // 8
HIST_WORDS = N_BATCH * HIST_W
WORDS_PER_TILE = HIST_WORDS // NS
ZCHUNK = 16000
QLAG = 24
TRASH = HIST_WORDS


def _hist_body(z_hbm, b_hbm, h0_hbm, h1_hbm, shared, zv, bv, fidx, vfull,
               zbuf, sem_in, sem_z, sem_s):
    c = lax.axis_index("c")
    s = lax.axis_index("s")
    wid = c * NS + s
    base = wid * ATOMS_PER_TILE

    pltpu.async_copy(z_hbm.at[pl.ds(base, ATOMS_PER_TILE)], zv, sem_in)
    pltpu.async_copy(b_hbm.at[pl.ds(base, ATOMS_PER_TILE)], bv, sem_in)

    zeros_f = jnp.zeros((16,), jnp.float32)
    ones_f = jnp.ones((16,), jnp.float32)
    trash_i = jnp.full((16,), TRASH, jnp.int32)

    def _zb(i, carry):
        for k in range(8):
            zbuf[pl.ds(i * 128 + k * 16, 16)] = zeros_f
        return carry
    lax.fori_loop(0, ZCHUNK // 128, _zb, 0)
    nz = WORDS_PER_TILE // ZCHUNK
    for k in range(nz):
        pltpu.async_copy(
            zbuf, shared.at[pl.ds(s * WORDS_PER_TILE + k * ZCHUNK, ZCHUNK)],
            sem_z)

    for k in range(8):
        vfull[pl.ds(k * 16, 16)] = ones_f

    pltpu.make_async_copy(z_hbm.at[pl.ds(base, ATOMS_PER_TILE)], zv,
                          sem_in).wait()
    pltpu.make_async_copy(b_hbm.at[pl.ds(base, ATOMS_PER_TILE)], bv,
                          sem_in).wait()

    full_rows = VECS // 8
    rem_vecs = VECS - full_rows * 8

    def _fb(r, carry):
        for k in range(8):
            off = r * 128 + k * 16
            zi = zv[pl.ds(off, 16)]
            bi = bv[pl.ds(off, 16)]
            fidx[r, pl.ds(k * 16, 16)] = bi * HIST_W + zi
        return carry
    lax.fori_loop(0, full_rows, _fb, 0)
    for k in range(8):
        if k < rem_vecs:
            off = full_rows * 128 + k * 16
            zi = zv[pl.ds(off, 16)]
            bi = bv[pl.ds(off, 16)]
            fidx[full_rows, pl.ds(k * 16, 16)] = bi * HIST_W + zi
        else:
            fidx[full_rows, pl.ds(k * 16, 16)] = trash_i

    for k in range(nz):
        pltpu.make_async_copy(
            zbuf, shared.at[pl.ds(s * WORDS_PER_TILE + k * ZCHUNK, ZCHUNK)],
            sem_z).wait()
    plsc.subcore_barrier()

    def _step(j, carry):
        @pl.when(j < IDX_ROWS)
        def _():
            pltpu.async_copy(vfull, shared.at[fidx.at[j]], sem_s, add=True)

        @pl.when(j >= QLAG)
        def _():
            pltpu.make_async_copy(vfull, shared.at[fidx.at[j - QLAG]],
                                  sem_s).wait()

        return carry
    lax.fori_loop(0, IDX_ROWS + QLAG, _step, 0)

    plsc.subcore_barrier()

    src = shared.at[pl.ds(s * WORDS_PER_TILE, WORDS_PER_TILE)]

    @pl.when(c == 0)
    def _():
        pltpu.sync_copy(src, h0_hbm.at[pl.ds(s * WORDS_PER_TILE,
                                             WORDS_PER_TILE)])

    @pl.when(c == 1)
    def _():
        pltpu.sync_copy(src, h1_hbm.at[pl.ds(s * WORDS_PER_TILE,
                                             WORDS_PER_TILE)])


_hist = functools.partial(
    pl.kernel,
    out_type=(jax.ShapeDtypeStruct((HIST_WORDS,), jnp.float32),
              jax.ShapeDtypeStruct((HIST_WORDS,), jnp.float32)),
    mesh=plsc.VectorSubcoreMesh(core_axis_name="c", subcore_axis_name="s"),
    scratch_types=[
        pltpu.VMEM_SHARED((HIST_WORDS + 128,), jnp.float32),
        pltpu.VMEM((ATOMS_PER_TILE,), jnp.int32),
        pltpu.VMEM((ATOMS_PER_TILE,), jnp.int32),
        pltpu.VMEM((IDX_ROWS, 128), jnp.int32),
        pltpu.VMEM((128,), jnp.float32),
        pltpu.VMEM((ZCHUNK,), jnp.float32),
        pltpu.SemaphoreType.DMA,
        pltpu.SemaphoreType.DMA,
        pltpu.SemaphoreType.DMA,
    ],
)(_hist_body)


ROWS_TC = 400
GRID_TC = N_BATCH // ROWS_TC


def _combine_body(h0_ref, h1_ref, out_ref, table_ref, mean_ref, o_ref):
    h = h0_ref[...] + h1_ref[...]
    table = table_ref[...] + mean_ref[...]
    o_ref[...] = out_ref[...] + jnp.dot(
        h, table, preferred_element_type=jnp.float32)


def _combine(h0, h1, out, table, mean2d):
    hspec = pl.BlockSpec((ROWS_TC, HIST_W), lambda i: (i, 0))
    return pl.pallas_call(
        _combine_body,
        grid=(GRID_TC,),
        in_specs=[
            hspec,
            hspec,
            pl.BlockSpec((ROWS_TC, OUT_DIM), lambda i: (i, 0)),
            pl.BlockSpec((HIST_W, OUT_DIM), lambda i: (0, 0)),
            pl.BlockSpec((1, OUT_DIM), lambda i: (0, 0)),
        ],
        out_specs=pl.BlockSpec((ROWS_TC, OUT_DIM), lambda i: (i, 0)),
        out_shape=jax.ShapeDtypeStruct((N_BATCH, OUT_DIM), jnp.float32),
    )(h0, h1, out, table, mean2d)


def kernel(out, z, batch_idx, atomref, mean):
    h0, h1 = _hist(z, batch_idx)
    h0 = h0.reshape(N_BATCH, HIST_W)
    h1 = h1.reshape(N_BATCH, HIST_W)
    table = jnp.pad(atomref, ((0, HIST_W - N_ATOMREF), (0, 0)))
    return _combine(h0, h1, out, table, mean.reshape(1, OUT_DIM))


# speedup vs baseline: 54.6473x; 1.1639x over previous
import functools

import jax
import jax.numpy as jnp
from jax import lax
from jax.experimental import pallas as pl
from jax.experimental.pallas import tpu as pltpu
from jax.experimental.pallas import tpu_sc as plsc

N_NODE = 320000
N_BATCH = 10000
OUT_DIM = 128
N_ATOMREF = 100
HIST_W = 128

NS = 16
NW = 2 * NS
ATOMS_PER_TILE = N_NODE // NW
VECS = ATOMS_PER_TILE // 16
IDX_ROWS = (VECS + 7) // 8
HIST_WORDS = N_BATCH * HIST_W
WORDS_PER_TILE = HIST_WORDS // NS
ZCHUNK = 16000
QLAG = 24
TRASH = HIST_WORDS


def _hist_body(z_hbm, b_hbm, h0_hbm, h1_hbm, shared, zv, bv, fidx, vfull,
               zbuf, sem_in, sem_z, sem_s):
    c = lax.axis_index("c")
    s = lax.axis_index("s")
    wid = c * NS + s
    base = wid * ATOMS_PER_TILE

    pltpu.async_copy(z_hbm.at[pl.ds(base, ATOMS_PER_TILE)], zv, sem_in)
    pltpu.async_copy(b_hbm.at[pl.ds(base, ATOMS_PER_TILE)], bv, sem_in)

    zeros_f = jnp.zeros((16,), jnp.float32)
    ones_f = jnp.ones((16,), jnp.float32)
    trash_i = jnp.full((16,), TRASH, jnp.int32)

    def _zb(i, carry):
        for k in range(8):
            zbuf[pl.ds(i * 128 + k * 16, 16)] = zeros_f
        return carry
    lax.fori_loop(0, ZCHUNK // 128, _zb, 0)
    nz = WORDS_PER_TILE // ZCHUNK
    for k in range(nz):
        pltpu.async_copy(
            zbuf, shared.at[pl.ds(s * WORDS_PER_TILE + k * ZCHUNK, ZCHUNK)],
            sem_z)

    for k in range(8):
        vfull[pl.ds(k * 16, 16)] = ones_f

    pltpu.make_async_copy(z_hbm.at[pl.ds(base, ATOMS_PER_TILE)], zv,
                          sem_in).wait()
    pltpu.make_async_copy(b_hbm.at[pl.ds(base, ATOMS_PER_TILE)], bv,
                          sem_in).wait()

    full_rows = VECS // 8
    rem_vecs = VECS - full_rows * 8

    def _fb(r, carry):
        for k in range(8):
            off = r * 128 + k * 16
            zi = zv[pl.ds(off, 16)]
            bi = bv[pl.ds(off, 16)]
            fidx[r, pl.ds(k * 16, 16)] = bi * HIST_W + zi
        return carry
    lax.fori_loop(0, full_rows, _fb, 0)
    for k in range(8):
        if k < rem_vecs:
            off = full_rows * 128 + k * 16
            zi = zv[pl.ds(off, 16)]
            bi = bv[pl.ds(off, 16)]
            fidx[full_rows, pl.ds(k * 16, 16)] = bi * HIST_W + zi
        else:
            fidx[full_rows, pl.ds(k * 16, 16)] = trash_i

    for k in range(nz):
        pltpu.make_async_copy(
            zbuf, shared.at[pl.ds(s * WORDS_PER_TILE + k * ZCHUNK, ZCHUNK)],
            sem_z).wait()
    plsc.subcore_barrier()

    def _step(j, carry):
        @pl.when(j < IDX_ROWS)
        def _():
            pltpu.async_copy(vfull, shared.at[fidx.at[j]], sem_s, add=True)

        @pl.when(j >= QLAG)
        def _():
            pltpu.make_async_copy(vfull, shared.at[fidx.at[j - QLAG]],
                                  sem_s).wait()

        return carry
    lax.fori_loop(0, IDX_ROWS + QLAG, _step, 0)

    plsc.subcore_barrier()

    src = shared.at[pl.ds(s * WORDS_PER_TILE, WORDS_PER_TILE)]

    @pl.when(c == 0)
    def _():
        pltpu.sync_copy(src, h0_hbm.at[pl.ds(s * WORDS_PER_TILE,
                                             WORDS_PER_TILE)])

    @pl.when(c == 1)
    def _():
        pltpu.sync_copy(src, h1_hbm.at[pl.ds(s * WORDS_PER_TILE,
                                             WORDS_PER_TILE)])


_hist = functools.partial(
    pl.kernel,
    out_type=(jax.ShapeDtypeStruct((HIST_WORDS,), jnp.float32),
              jax.ShapeDtypeStruct((HIST_WORDS,), jnp.float32)),
    mesh=plsc.VectorSubcoreMesh(core_axis_name="c", subcore_axis_name="s"),
    scratch_types=[
        pltpu.VMEM_SHARED((HIST_WORDS + 128,), jnp.float32),
        pltpu.VMEM((ATOMS_PER_TILE,), jnp.int32),
        pltpu.VMEM((ATOMS_PER_TILE,), jnp.int32),
        pltpu.VMEM((IDX_ROWS, 128), jnp.int32),
        pltpu.VMEM((128,), jnp.float32),
        pltpu.VMEM((ZCHUNK,), jnp.float32),
        pltpu.SemaphoreType.DMA,
        pltpu.SemaphoreType.DMA,
        pltpu.SemaphoreType.DMA,
    ],
)(_hist_body)


ROWS_TC = 1000
GRID_TC = N_BATCH // ROWS_TC


def _combine_body(h0_ref, h1_ref, out_ref, table_ref, mean_ref, o_ref):
    h = h0_ref[...] + h1_ref[...]
    table = table_ref[...] + mean_ref[...]
    o_ref[...] = out_ref[...] + jnp.dot(
        h, table, preferred_element_type=jnp.float32)


def _combine(h0, h1, out, table, mean2d):
    hspec = pl.BlockSpec((ROWS_TC, HIST_W), lambda i: (i, 0))
    return pl.pallas_call(
        _combine_body,
        grid=(GRID_TC,),
        in_specs=[
            hspec,
            hspec,
            pl.BlockSpec((ROWS_TC, OUT_DIM), lambda i: (i, 0)),
            pl.BlockSpec((HIST_W, OUT_DIM), lambda i: (0, 0)),
            pl.BlockSpec((1, OUT_DIM), lambda i: (0, 0)),
        ],
        out_specs=pl.BlockSpec((ROWS_TC, OUT_DIM), lambda i: (i, 0)),
        out_shape=jax.ShapeDtypeStruct((N_BATCH, OUT_DIM), jnp.float32),
    )(h0, h1, out, table, mean2d)


def kernel(out, z, batch_idx, atomref, mean):
    h0, h1 = _hist(z, batch_idx)
    h0 = h0.reshape(N_BATCH, HIST_W)
    h1 = h1.reshape(N_BATCH, HIST_W)
    table = jnp.pad(atomref, ((0, HIST_W - N_ATOMREF), (0, 0)))
    return _combine(h0, h1, out, table, mean.reshape(1, OUT_DIM))


# speedup vs baseline: 57.9210x; 1.0599x over previous
import functools

import jax
import jax.numpy as jnp
from jax import lax
from jax.experimental import pallas as pl
from jax.experimental.pallas import tpu as pltpu
from jax.experimental.pallas import tpu_sc as plsc

N_NODE = 320000
N_BATCH = 10000
OUT_DIM = 128
N_ATOMREF = 100
HIST_W = 128

NS = 16
NW = 2 * NS
ATOMS_PER_TILE = N_NODE // NW
VECS = ATOMS_PER_TILE // 16
IDX_ROWS = (VECS + 7) // 8
HIST_WORDS = N_BATCH * HIST_W
WORDS_PER_TILE = HIST_WORDS // NS
ZCHUNK = 16000
QLAG = 24
TRASH = HIST_WORDS


def _hist_body(z_hbm, b_hbm, h0_hbm, h1_hbm, shared, zv, bv, fidx, vfull,
               zbuf, sem_in, sem_z, sem_s):
    c = lax.axis_index("c")
    s = lax.axis_index("s")
    wid = c * NS + s
    base = wid * ATOMS_PER_TILE

    pltpu.async_copy(z_hbm.at[pl.ds(base, ATOMS_PER_TILE)], zv, sem_in)
    pltpu.async_copy(b_hbm.at[pl.ds(base, ATOMS_PER_TILE)], bv, sem_in)

    zeros_f = jnp.zeros((16,), jnp.float32)
    ones_f = jnp.ones((16,), jnp.float32)
    trash_i = jnp.full((16,), TRASH, jnp.int32)

    def _zb(i, carry):
        for k in range(8):
            zbuf[pl.ds(i * 128 + k * 16, 16)] = zeros_f
        return carry
    lax.fori_loop(0, ZCHUNK // 128, _zb, 0)
    nz = WORDS_PER_TILE // ZCHUNK
    for k in range(nz):
        pltpu.async_copy(
            zbuf, shared.at[pl.ds(s * WORDS_PER_TILE + k * ZCHUNK, ZCHUNK)],
            sem_z)

    for k in range(8):
        vfull[pl.ds(k * 16, 16)] = ones_f

    pltpu.make_async_copy(z_hbm.at[pl.ds(base, ATOMS_PER_TILE)], zv,
                          sem_in).wait()
    pltpu.make_async_copy(b_hbm.at[pl.ds(base, ATOMS_PER_TILE)], bv,
                          sem_in).wait()

    full_rows = VECS // 8
    rem_vecs = VECS - full_rows * 8

    def _fb(r, carry):
        for k in range(8):
            off = r * 128 + k * 16
            zi = zv[pl.ds(off, 16)]
            bi = bv[pl.ds(off, 16)]
            fidx[r, pl.ds(k * 16, 16)] = bi * HIST_W + zi
        return carry
    lax.fori_loop(0, full_rows, _fb, 0)
    for k in range(8):
        if k < rem_vecs:
            off = full_rows * 128 + k * 16
            zi = zv[pl.ds(off, 16)]
            bi = bv[pl.ds(off, 16)]
            fidx[full_rows, pl.ds(k * 16, 16)] = bi * HIST_W + zi
        else:
            fidx[full_rows, pl.ds(k * 16, 16)] = trash_i

    for k in range(nz):
        pltpu.make_async_copy(
            zbuf, shared.at[pl.ds(s * WORDS_PER_TILE + k * ZCHUNK, ZCHUNK)],
            sem_z).wait()
    plsc.subcore_barrier()

    def _step(j, carry):
        @pl.when(j < IDX_ROWS)
        def _():
            pltpu.async_copy(vfull, shared.at[fidx.at[j]], sem_s, add=True)

        @pl.when(j >= QLAG)
        def _():
            pltpu.make_async_copy(vfull, shared.at[fidx.at[j - QLAG]],
                                  sem_s).wait()

        return carry
    lax.fori_loop(0, IDX_ROWS + QLAG, _step, 0)

    plsc.subcore_barrier()

    src = shared.at[pl.ds(s * WORDS_PER_TILE, WORDS_PER_TILE)]

    @pl.when(c == 0)
    def _():
        pltpu.sync_copy(src, h0_hbm.at[pl.ds(s * WORDS_PER_TILE,
                                             WORDS_PER_TILE)])

    @pl.when(c == 1)
    def _():
        pltpu.sync_copy(src, h1_hbm.at[pl.ds(s * WORDS_PER_TILE,
                                             WORDS_PER_TILE)])


_hist = functools.partial(
    pl.kernel,
    out_type=(jax.ShapeDtypeStruct((HIST_WORDS,), jnp.float32),
              jax.ShapeDtypeStruct((HIST_WORDS,), jnp.float32)),
    mesh=plsc.VectorSubcoreMesh(core_axis_name="c", subcore_axis_name="s"),
    scratch_types=[
        pltpu.VMEM_SHARED((HIST_WORDS + 128,), jnp.float32),
        pltpu.VMEM((ATOMS_PER_TILE,), jnp.int32),
        pltpu.VMEM((ATOMS_PER_TILE,), jnp.int32),
        pltpu.VMEM((IDX_ROWS, 128), jnp.int32),
        pltpu.VMEM((128,), jnp.float32),
        pltpu.VMEM((ZCHUNK,), jnp.float32),
        pltpu.SemaphoreType.DMA,
        pltpu.SemaphoreType.DMA,
        pltpu.SemaphoreType.DMA,
    ],
)(_hist_body)


ROWS_TC = 2000
GRID_TC = N_BATCH // ROWS_TC


def _combine_body(h0_ref, h1_ref, out_ref, table_ref, mean_ref, o_ref):
    h = h0_ref[...] + h1_ref[...]
    table = table_ref[...] + mean_ref[...]
    o_ref[...] = out_ref[...] + jnp.dot(
        h, table, preferred_element_type=jnp.float32)


def _combine(h0, h1, out, table, mean2d):
    hspec = pl.BlockSpec((ROWS_TC, HIST_W), lambda i: (i, 0))
    return pl.pallas_call(
        _combine_body,
        grid=(GRID_TC,),
        in_specs=[
            hspec,
            hspec,
            pl.BlockSpec((ROWS_TC, OUT_DIM), lambda i: (i, 0)),
            pl.BlockSpec((HIST_W, OUT_DIM), lambda i: (0, 0)),
            pl.BlockSpec((1, OUT_DIM), lambda i: (0, 0)),
        ],
        out_specs=pl.BlockSpec((ROWS_TC, OUT_DIM), lambda i: (i, 0)),
        out_shape=jax.ShapeDtypeStruct((N_BATCH, OUT_DIM), jnp.float32),
    )(h0, h1, out, table, mean2d)


def kernel(out, z, batch_idx, atomref, mean):
    h0, h1 = _hist(z, batch_idx)
    h0 = h0.reshape(N_BATCH, HIST_W)
    h1 = h1.reshape(N_BATCH, HIST_W)
    table = jnp.pad(atomref, ((0, HIST_W - N_ATOMREF), (0, 0)))
    return _combine(h0, h1, out, table, mean.reshape(1, OUT_DIM))
